# baseline (speedup 1.0000x reference)
import jax
import jax.numpy as jnp
from jax import lax
from jax.experimental import pallas as pl
from jax.experimental.pallas import tpu as pltpu
from jax.experimental.pallas import tpu_sc as plsc

D = 32
B = 16384
H = 50
N = B * H
NC, NS = 2, 16
NW = NC * NS
SB = B // NW
BLK = 10
NBLK = H // BLK


def _gather_body(idx_hbm, table_hbm, out_hbm, idx_v, rows_v,
                 gsem0, gsem1, ssem0, ssem1):
    wid = lax.axis_index("s") * NC + lax.axis_index("c")
    b0 = wid * SB
    gsem = (gsem0, gsem1)
    ssem = (ssem0, ssem1)

    pltpu.sync_copy(idx_hbm.at[:, pl.ds(b0, SB)], idx_v)

    def block(g, carry):
        stores = [None, None]
        gathers = [None, None]
        for j in range(BLK):
            c = g * BLK + j
            s = j % 2
            if stores[s] is not None:
                stores[s].wait()
            gathers[s] = pltpu.make_async_copy(
                table_hbm.at[idx_v.at[c]], rows_v.at[s], gsem[s])
            gathers[s].start()
            if j > 0:
                p = (j - 1) % 2
                gathers[p].wait()
                stores[p] = pltpu.make_async_copy(
                    rows_v.at[p], out_hbm.at[pl.ds((c - 1) * B + b0, SB)],
                    ssem[p])
                stores[p].start()
        last = (BLK - 1) % 2
        gathers[last].wait()
        stores[last] = pltpu.make_async_copy(
            rows_v.at[last],
            out_hbm.at[pl.ds((g * BLK + BLK - 1) * B + b0, SB)], ssem[last])
        stores[last].start()
        stores[1 - last].wait()
        stores[last].wait()
        return carry

    lax.fori_loop(0, NBLK, block, 0)


@jax.jit
def _gather(idx_t, table):
    f = pl.kernel(
        _gather_body,
        out_type=jax.ShapeDtypeStruct((N, D), jnp.float32),
        mesh=plsc.VectorSubcoreMesh(core_axis_name="c", subcore_axis_name="s"),
        scratch_types=[
            pltpu.VMEM((H, SB), jnp.int32),
            pltpu.VMEM((2, SB, D), jnp.float32),
            pltpu.SemaphoreType.DMA,
            pltpu.SemaphoreType.DMA,
            pltpu.SemaphoreType.DMA,
            pltpu.SemaphoreType.DMA,
        ],
        compiler_params=pltpu.CompilerParams(use_tc_tiling_on_sc=False),
    )
    return f(idx_t, table)


def kernel(input, embeddings):
    idx_t = input.T.astype(jnp.int32)
    out = _gather(idx_t, embeddings)
    return out.reshape(H, B, D).transpose(1, 0, 2)


# speedup vs baseline: 1.0014x; 1.0014x over previous
import jax
import jax.numpy as jnp
from jax import lax
from jax.experimental import pallas as pl
from jax.experimental.pallas import tpu as pltpu
from jax.experimental.pallas import tpu_sc as plsc

D = 32
B = 16384
H = 50
N = B * H
NC, NS = 2, 16
NW = NC * NS
SB = B // NW
PER_W = N // NW
CHUNK = 1600
NCHUNK = PER_W // CHUNK


def _format_body(idx_hbm, out_hbm, idx_v):
    wid = lax.axis_index("s") * NC + lax.axis_index("c")
    b0 = wid * SB
    pltpu.sync_copy(idx_hbm.at[:, pl.ds(b0, SB)], idx_v)
    for h in range(H):
        pltpu.sync_copy(idx_v.at[h], out_hbm.at[pl.ds(h * B + b0, SB)])


@jax.jit
def _format_idx(idx_t):
    f = pl.kernel(
        _format_body,
        out_type=jax.ShapeDtypeStruct((N,), jnp.int32),
        mesh=plsc.VectorSubcoreMesh(core_axis_name="c", subcore_axis_name="s"),
        scratch_types=[
            pltpu.VMEM((H, SB), jnp.int32),
        ],
        compiler_params=pltpu.CompilerParams(use_tc_tiling_on_sc=True),
    )
    return f(idx_t)


def _gather_body(idx_hbm, table_hbm, out_hbm, idx_v, rows_v,
                 gsem0, gsem1, ssem0, ssem1):
    wid = lax.axis_index("s") * NC + lax.axis_index("c")
    base = wid * PER_W
    gsem = (gsem0, gsem1)
    ssem = (ssem0, ssem1)

    pltpu.sync_copy(idx_hbm.at[pl.ds(base, PER_W)], idx_v)

    stores = [None, None]
    gathers = [None, None]
    for c in range(NCHUNK):
        s = c % 2
        if stores[s] is not None:
            stores[s].wait()
        gathers[s] = pltpu.make_async_copy(
            table_hbm.at[idx_v.at[pl.ds(c * CHUNK, CHUNK)]], rows_v.at[s],
            gsem[s])
        gathers[s].start()
        if c > 0:
            p = (c - 1) % 2
            gathers[p].wait()
            stores[p] = pltpu.make_async_copy(
                rows_v.at[p], out_hbm.at[pl.ds(base + (c - 1) * CHUNK, CHUNK)],
                ssem[p])
            stores[p].start()
    last = (NCHUNK - 1) % 2
    gathers[last].wait()
    stores[last] = pltpu.make_async_copy(
        rows_v.at[last],
        out_hbm.at[pl.ds(base + (NCHUNK - 1) * CHUNK, CHUNK)], ssem[last])
    stores[last].start()
    stores[1 - last].wait()
    stores[last].wait()


@jax.jit
def _gather(idx_flat, table):
    f = pl.kernel(
        _gather_body,
        out_type=jax.ShapeDtypeStruct((N, D), jnp.float32),
        mesh=plsc.VectorSubcoreMesh(core_axis_name="c", subcore_axis_name="s"),
        scratch_types=[
            pltpu.VMEM((PER_W,), jnp.int32),
            pltpu.VMEM((2, CHUNK, D), jnp.float32),
            pltpu.SemaphoreType.DMA,
            pltpu.SemaphoreType.DMA,
            pltpu.SemaphoreType.DMA,
            pltpu.SemaphoreType.DMA,
        ],
        compiler_params=pltpu.CompilerParams(use_tc_tiling_on_sc=False),
    )
    return f(idx_flat, table)


def kernel(input, embeddings):
    idx_flat = _format_idx(input.T.astype(jnp.int32))
    out = _gather(idx_flat, embeddings)
    return out.reshape(H, B, D).transpose(1, 0, 2)
